# baseline (speedup 1.0000x reference)
import functools

import jax
import jax.numpy as jnp
from jax import lax
from jax.experimental import pallas as pl
from jax.experimental.pallas import tpu as pltpu
from jax.experimental.pallas import tpu_sc as plsc

NUM_CORES = 2
NUM_SUBCORES = 16
NUM_WORKERS = NUM_CORES * NUM_SUBCORES
CHUNK = 128


@functools.partial(jax.jit, static_argnums=(2, 3))
def _sc_gather(weight, flat_idx, b_per_w, n_chunks):
    d = weight.shape[1]
    total = flat_idx.shape[0]
    mesh = plsc.VectorSubcoreMesh(core_axis_name="c", subcore_axis_name="s")

    @functools.partial(
        pl.kernel,
        mesh=mesh,
        out_type=jax.ShapeDtypeStruct((total, d), jnp.float32),
        scratch_types=[
            pltpu.VMEM((b_per_w,), jnp.int32),
            pltpu.VMEM((CHUNK, d), jnp.float32),
            pltpu.SemaphoreType.DMA,
        ],
        compiler_params=pltpu.CompilerParams(use_tc_tiling_on_sc=False),
    )
    def k(table_hbm, idx_hbm, out_hbm, idx_v, rows_v, sem):
        wid = lax.axis_index("s") * NUM_CORES + lax.axis_index("c")
        base = wid * b_per_w
        pltpu.sync_copy(idx_hbm.at[pl.ds(base, b_per_w)], idx_v)

        def body(i, carry):
            off = i * CHUNK
            idx_slice = idx_v.at[pl.ds(off, CHUNK)]
            pltpu.async_copy(table_hbm.at[idx_slice], rows_v, sem).wait()
            pltpu.sync_copy(rows_v, out_hbm.at[pl.ds(base + off, CHUNK)])
            return carry

        lax.fori_loop(0, n_chunks, body, 0)

    return k(weight, flat_idx)


def kernel(token_ids, weight):
    b = token_ids.size
    flat = token_ids.reshape((b,)).astype(jnp.int32)
    b_per_w = b // NUM_WORKERS
    n_chunks = b_per_w // CHUNK
    out = _sc_gather(weight, flat, b_per_w, n_chunks)
    return out.reshape(token_ids.shape + (weight.shape[1],))


# speedup vs baseline: 1.1147x; 1.1147x over previous
import functools

import jax
import jax.numpy as jnp
from jax import lax
from jax.experimental import pallas as pl
from jax.experimental.pallas import tpu as pltpu
from jax.experimental.pallas import tpu_sc as plsc

NUM_CORES = 2
NUM_SUBCORES = 16
NUM_WORKERS = NUM_CORES * NUM_SUBCORES
CHUNK = 128
RING = 8
LAG = 4


@functools.partial(jax.jit, static_argnums=(2, 3))
def _sc_gather(weight, flat_idx, b_per_w, n_chunks):
    d = weight.shape[1]
    total = flat_idx.shape[0]
    mesh = plsc.VectorSubcoreMesh(core_axis_name="c", subcore_axis_name="s")

    @functools.partial(
        pl.kernel,
        mesh=mesh,
        out_type=jax.ShapeDtypeStruct((total, d), jnp.float32),
        scratch_types=[
            pltpu.VMEM((b_per_w,), jnp.int32),
            pltpu.VMEM((RING, CHUNK, d), jnp.float32),
            pltpu.SemaphoreType.DMA((RING,)),
            pltpu.SemaphoreType.DMA((RING,)),
        ],
        compiler_params=pltpu.CompilerParams(use_tc_tiling_on_sc=False),
    )
    def k(table_hbm, idx_hbm, out_hbm, idx_v, rows_v, gsem, wsem):
        wid = lax.axis_index("s") * NUM_CORES + lax.axis_index("c")
        base = wid * b_per_w
        pltpu.sync_copy(idx_hbm.at[pl.ds(base, b_per_w)], idx_v)

        def start_gather(c, s):
            pltpu.async_copy(
                table_hbm.at[idx_v.at[pl.ds(c * CHUNK, CHUNK)]],
                rows_v.at[s],
                gsem.at[s],
            )

        def wait_gather(c, s):
            pltpu.make_async_copy(
                table_hbm.at[idx_v.at[pl.ds(c * CHUNK, CHUNK)]],
                rows_v.at[s],
                gsem.at[s],
            ).wait()

        def start_write(c, s):
            pltpu.async_copy(
                rows_v.at[s],
                out_hbm.at[pl.ds(base + c * CHUNK, CHUNK)],
                wsem.at[s],
            )

        def wait_write(c, s):
            pltpu.make_async_copy(
                rows_v.at[s],
                out_hbm.at[pl.ds(base + c * CHUNK, CHUNK)],
                wsem.at[s],
            ).wait()

        for i in range(RING):
            start_gather(i, i)
            if i >= LAG:
                j = i - LAG
                wait_gather(j, j)
                start_write(j, j)

        def body(kk, carry):
            i0 = kk * RING
            for b in range(RING):
                sw = (b - LAG) % RING
                wait_write(i0 + b - RING, b)
                start_gather(i0 + b, b)
                wait_gather(i0 + b - LAG, sw)
                start_write(i0 + b - LAG, sw)
            return carry

        lax.fori_loop(1, n_chunks // RING, body, 0)

        for b in range(LAG):
            c = n_chunks - LAG + b
            s = c % RING
            wait_gather(c, s)
            start_write(c, s)
        for b in range(RING):
            wait_write(n_chunks - RING + b, (n_chunks - RING + b) % RING)

    return k(weight, flat_idx)


def kernel(token_ids, weight):
    b = token_ids.size
    flat = token_ids.reshape((b,)).astype(jnp.int32)
    b_per_w = b // NUM_WORKERS
    n_chunks = b_per_w // CHUNK
    out = _sc_gather(weight, flat, b_per_w, n_chunks)
    return out.reshape(token_ids.shape + (weight.shape[1],))


# speedup vs baseline: 1.1799x; 1.0585x over previous
import functools

import jax
import jax.numpy as jnp
from jax import lax
from jax.experimental import pallas as pl
from jax.experimental.pallas import tpu as pltpu
from jax.experimental.pallas import tpu_sc as plsc

NUM_CORES = 2
NUM_SUBCORES = 16
NUM_WORKERS = NUM_CORES * NUM_SUBCORES

V = 1000000
D = 64
VMAIN = 999936
NCOLS = 7812
COLS_PER_TILE = 245

CHUNK = 128
RING = 4
LAG = 2


def _iota16():
    return lax.iota(jnp.int32, 16)


def _transpose16(src_ref, dst_ref, s0, d0):
    lanes = _iota16()
    for dd in range(16):
        rot = (lanes + dd) & 15
        x = plsc.load_gather(src_ref, [d0 + rot, s0 + lanes])
        plsc.store_scatter(dst_ref, [s0 + lanes, d0 + rot], x)


def _transpose_block(src_ref, dst_ref, n_s, n_d):
    nd16 = n_d // 16

    def sub(t, carry):
        s0 = (t // nd16) * 16
        d0 = (t % nd16) * 16
        _transpose16(src_ref, dst_ref, s0, d0)
        return carry

    lax.fori_loop(0, (n_s // 16) * nd16, sub, 0)


@jax.jit
def _sc_transpose(wt, tail):
    mesh = plsc.VectorSubcoreMesh(core_axis_name="c", subcore_axis_name="s")

    @functools.partial(
        pl.kernel,
        mesh=mesh,
        out_type=jax.ShapeDtypeStruct((V, 128), jnp.float32),
        scratch_types=[
            pltpu.VMEM((2, D, 128), jnp.float32),
            pltpu.VMEM((2, 128, 128), jnp.float32),
            pltpu.VMEM((D, D), jnp.float32),
            pltpu.SemaphoreType.DMA((2,)),
            pltpu.SemaphoreType.DMA((2,)),
        ],
        compiler_params=pltpu.CompilerParams(use_tc_tiling_on_sc=True, needs_layout_passes=False),
    )
    def k(wt_hbm, tail_hbm, out_hbm, blk_v, obuf_v, tail_v, isem, osem):
        wid = lax.axis_index("s") * NUM_CORES + lax.axis_index("c")

        def col_of(kk):
            return jnp.minimum(wid + 32 * kk, NCOLS - 1)

        def fire_in(c, p):
            for r in range(8):
                pltpu.async_copy(
                    wt_hbm.at[pl.ds(r * 8, 8), pl.ds(c * 128, 128)],
                    blk_v.at[p, pl.ds(r * 8, 8)],
                    isem.at[p],
                )

        def wait_in(c, p):
            for r in range(8):
                pltpu.make_async_copy(
                    wt_hbm.at[pl.ds(r * 8, 8), pl.ds(c * 128, 128)],
                    blk_v.at[p, pl.ds(r * 8, 8)],
                    isem.at[p],
                ).wait()

        def fire_out(c, p):
            pltpu.async_copy(
                obuf_v.at[p], out_hbm.at[pl.ds(c * 128, 128)], osem.at[p]
            )

        def wait_out(c, p):
            pltpu.make_async_copy(
                obuf_v.at[p], out_hbm.at[pl.ds(c * 128, 128)], osem.at[p]
            ).wait()

        def do_transpose(p):
            _transpose_block(blk_v.at[p], obuf_v.at[p], 128, D)

        fire_in(col_of(0), 0)
        fire_in(col_of(1), 1)

        def body(kk, carry):
            for p in range(2):
                c = col_of(2 * kk + p)
                wait_in(c, p)
                do_transpose(p)
                nxt = col_of(2 * kk + p + 2)
                fire_out(c, p)
                wait_out(c, p)
                fire_in(nxt, p)
            return carry

        lax.fori_loop(0, 122, body, 0)
        for p in range(2):
            c = col_of(244 + p)
            wait_in(c, p)
            do_transpose(p)
            fire_out(c, p)
            wait_out(c, p)

        pltpu.sync_copy(tail_hbm, tail_v)
        _transpose_block(tail_v, obuf_v.at[0], D, D)
        pltpu.sync_copy(
            obuf_v.at[0, pl.ds(0, D)], out_hbm.at[pl.ds(VMAIN, D)]
        )

    return k(wt, tail)


@functools.partial(jax.jit, static_argnums=(2, 3))
def _sc_gather(table, flat_idx, b_per_w, n_chunks):
    total = flat_idx.shape[0]
    mesh = plsc.VectorSubcoreMesh(core_axis_name="c", subcore_axis_name="s")

    @functools.partial(
        pl.kernel,
        mesh=mesh,
        out_type=jax.ShapeDtypeStruct((total, 128), jnp.float32),
        scratch_types=[
            pltpu.VMEM((b_per_w,), jnp.int32),
            pltpu.VMEM((RING, CHUNK, 128), jnp.float32),
            pltpu.SemaphoreType.DMA((RING,)),
            pltpu.SemaphoreType.DMA((RING,)),
        ],
        compiler_params=pltpu.CompilerParams(use_tc_tiling_on_sc=True, needs_layout_passes=False),
    )
    def k(table_hbm, idx_hbm, out_hbm, idx_v, rows_v, gsem, wsem):
        wid = lax.axis_index("s") * NUM_CORES + lax.axis_index("c")
        base = wid * b_per_w
        pltpu.sync_copy(idx_hbm.at[pl.ds(base, b_per_w)], idx_v)

        def start_gather(c, s):
            pltpu.async_copy(
                table_hbm.at[idx_v.at[pl.ds(c * CHUNK, CHUNK)]],
                rows_v.at[s],
                gsem.at[s],
            )

        def wait_gather(c, s):
            pltpu.make_async_copy(
                table_hbm.at[idx_v.at[pl.ds(c * CHUNK, CHUNK)]],
                rows_v.at[s],
                gsem.at[s],
            ).wait()

        def start_write(c, s):
            pltpu.async_copy(
                rows_v.at[s],
                out_hbm.at[pl.ds(base + c * CHUNK, CHUNK)],
                wsem.at[s],
            )

        def wait_write(c, s):
            pltpu.make_async_copy(
                rows_v.at[s],
                out_hbm.at[pl.ds(base + c * CHUNK, CHUNK)],
                wsem.at[s],
            ).wait()

        for i in range(RING):
            start_gather(i, i)
            if i >= LAG:
                j = i - LAG
                wait_gather(j, j)
                start_write(j, j)

        def body(kk, carry):
            i0 = kk * RING
            for b in range(RING):
                sw = (b - LAG) % RING
                wait_write(i0 + b - RING, b)
                start_gather(i0 + b, b)
                wait_gather(i0 + b - LAG, sw)
                start_write(i0 + b - LAG, sw)
            return carry

        lax.fori_loop(1, n_chunks // RING, body, 0)

        for b in range(LAG):
            c = n_chunks - LAG + b
            s = c % RING
            wait_gather(c, s)
            start_write(c, s)
        for b in range(RING):
            wait_write(n_chunks - RING + b, (n_chunks - RING + b) % RING)

    return k(table, flat_idx)


def kernel(token_ids, weight):
    b = token_ids.size
    flat = token_ids.reshape((b,)).astype(jnp.int32)
    wt = weight.T
    tail = weight[VMAIN:].T
    table = _sc_transpose(wt, tail)
    b_per_w = b // NUM_WORKERS
    n_chunks = b_per_w // CHUNK
    y = _sc_gather(table, flat, b_per_w, n_chunks)
    return y[:, :D].reshape(token_ids.shape + (D,))


# speedup vs baseline: 1.2229x; 1.0364x over previous
import functools

import jax
import jax.numpy as jnp
from jax import lax
from jax.experimental import pallas as pl
from jax.experimental.pallas import tpu as pltpu
from jax.experimental.pallas import tpu_sc as plsc

NUM_CORES = 2
NUM_SUBCORES = 16
NUM_WORKERS = NUM_CORES * NUM_SUBCORES

V = 1000000
D = 64
VMAIN = 999936
NCOLS = 7812
COLS_PER_TILE = 245

CHUNK = 128
RING = 4
LAG = 2


def _iota16():
    return lax.iota(jnp.int32, 16)


def _diag_vectors():
    lanes = _iota16()
    return [(lanes + dd) & 15 for dd in range(16)], lanes


def _transpose_block(src_ref, dst_ref, n_s, n_d, rots, lanes):
    def srow(si, carry):
        svec = si * 16 + lanes
        for di in range(n_d // 16):
            for dd in range(16):
                dvec = di * 16 + rots[dd]
                x = plsc.load_gather(src_ref, [dvec, svec])
                plsc.store_scatter(dst_ref, [svec, dvec], x)
        return carry

    lax.fori_loop(0, n_s // 16, srow, 0)


@jax.jit
def _sc_transpose(wt, tail):
    mesh = plsc.VectorSubcoreMesh(core_axis_name="c", subcore_axis_name="s")

    @functools.partial(
        pl.kernel,
        mesh=mesh,
        out_type=jax.ShapeDtypeStruct((V, 128), jnp.float32),
        scratch_types=[
            pltpu.VMEM((2, D, 128), jnp.float32),
            pltpu.VMEM((2, 128, 128), jnp.float32),
            pltpu.VMEM((D, D), jnp.float32),
            pltpu.SemaphoreType.DMA((2,)),
            pltpu.SemaphoreType.DMA((2,)),
        ],
        compiler_params=pltpu.CompilerParams(use_tc_tiling_on_sc=True, needs_layout_passes=False),
    )
    def k(wt_hbm, tail_hbm, out_hbm, blk_v, obuf_v, tail_v, isem, osem):
        wid = lax.axis_index("s") * NUM_CORES + lax.axis_index("c")
        rots, lanes0 = _diag_vectors()

        def col_of(kk):
            return jnp.minimum(wid + 32 * kk, NCOLS - 1)

        def fire_in(c, p):
            for r in range(8):
                pltpu.async_copy(
                    wt_hbm.at[pl.ds(r * 8, 8), pl.ds(c * 128, 128)],
                    blk_v.at[p, pl.ds(r * 8, 8)],
                    isem.at[p],
                )

        def wait_in(c, p):
            for r in range(8):
                pltpu.make_async_copy(
                    wt_hbm.at[pl.ds(r * 8, 8), pl.ds(c * 128, 128)],
                    blk_v.at[p, pl.ds(r * 8, 8)],
                    isem.at[p],
                ).wait()

        def fire_out(c, p):
            pltpu.async_copy(
                obuf_v.at[p], out_hbm.at[pl.ds(c * 128, 128)], osem.at[p]
            )

        def wait_out(c, p):
            pltpu.make_async_copy(
                obuf_v.at[p], out_hbm.at[pl.ds(c * 128, 128)], osem.at[p]
            ).wait()

        fire_in(col_of(0), 0)
        fire_in(col_of(1), 1)

        def body(kk, carry):
            p = kk & 1
            c = col_of(kk)
            wait_in(c, p)
            pl.when(kk >= 2)(lambda: wait_out(col_of(kk - 2), p))
            _transpose_block(blk_v.at[p], obuf_v.at[p], 128, D, rots, lanes0)
            fire_out(c, p)
            pl.when(kk <= 242)(lambda: fire_in(col_of(kk + 2), p))
            return carry

        lax.fori_loop(0, 245, body, 0)
        wait_out(col_of(243), 1)
        wait_out(col_of(244), 0)

        pltpu.sync_copy(tail_hbm, tail_v)

        def tail_sub(t, carry):
            svec = (t // 4) * 16 + lanes0
            for dd in range(16):
                dvec = (t % 4) * 16 + rots[dd]
                x = plsc.load_gather(tail_v, [dvec, svec])
                plsc.store_scatter(obuf_v.at[0], [svec, dvec], x)
            return carry

        lax.fori_loop(0, 16, tail_sub, 0)
        pltpu.sync_copy(
            obuf_v.at[0, pl.ds(0, D)], out_hbm.at[pl.ds(VMAIN, D)]
        )

    return k(wt, tail)


@functools.partial(jax.jit, static_argnums=(2, 3))
def _sc_gather(table, flat_idx, b_per_w, n_chunks):
    total = flat_idx.shape[0]
    mesh = plsc.VectorSubcoreMesh(core_axis_name="c", subcore_axis_name="s")

    @functools.partial(
        pl.kernel,
        mesh=mesh,
        out_type=jax.ShapeDtypeStruct((total, 128), jnp.float32),
        scratch_types=[
            pltpu.VMEM((b_per_w,), jnp.int32),
            pltpu.VMEM((RING, CHUNK, 128), jnp.float32),
            pltpu.SemaphoreType.DMA((RING,)),
            pltpu.SemaphoreType.DMA((RING,)),
        ],
        compiler_params=pltpu.CompilerParams(use_tc_tiling_on_sc=True, needs_layout_passes=False),
    )
    def k(table_hbm, idx_hbm, out_hbm, idx_v, rows_v, gsem, wsem):
        wid = lax.axis_index("s") * NUM_CORES + lax.axis_index("c")
        base = wid * b_per_w
        pltpu.sync_copy(idx_hbm.at[pl.ds(base, b_per_w)], idx_v)

        def start_gather(c, s):
            pltpu.async_copy(
                table_hbm.at[idx_v.at[pl.ds(c * CHUNK, CHUNK)]],
                rows_v.at[s],
                gsem.at[s],
            )

        def wait_gather(c, s):
            pltpu.make_async_copy(
                table_hbm.at[idx_v.at[pl.ds(c * CHUNK, CHUNK)]],
                rows_v.at[s],
                gsem.at[s],
            ).wait()

        def start_write(c, s):
            pltpu.async_copy(
                rows_v.at[s],
                out_hbm.at[pl.ds(base + c * CHUNK, CHUNK)],
                wsem.at[s],
            )

        def wait_write(c, s):
            pltpu.make_async_copy(
                rows_v.at[s],
                out_hbm.at[pl.ds(base + c * CHUNK, CHUNK)],
                wsem.at[s],
            ).wait()

        for i in range(RING):
            start_gather(i, i)
            if i >= LAG:
                j = i - LAG
                wait_gather(j, j)
                start_write(j, j)

        def body(kk, carry):
            i0 = kk * RING
            for b in range(RING):
                sw = (b - LAG) % RING
                wait_write(i0 + b - RING, b)
                start_gather(i0 + b, b)
                wait_gather(i0 + b - LAG, sw)
                start_write(i0 + b - LAG, sw)
            return carry

        lax.fori_loop(1, n_chunks // RING, body, 0)

        for b in range(LAG):
            c = n_chunks - LAG + b
            s = c % RING
            wait_gather(c, s)
            start_write(c, s)
        for b in range(RING):
            wait_write(n_chunks - RING + b, (n_chunks - RING + b) % RING)

    return k(table, flat_idx)


def kernel(token_ids, weight):
    b = token_ids.size
    flat = token_ids.reshape((b,)).astype(jnp.int32)
    wt = weight.T
    tail = weight[VMAIN:].T
    table = _sc_transpose(wt, tail)
    b_per_w = b // NUM_WORKERS
    n_chunks = b_per_w // CHUNK
    y = _sc_gather(table, flat, b_per_w, n_chunks)
    return y[:, :D].reshape(token_ids.shape + (D,))


# speedup vs baseline: 1.7188x; 1.4056x over previous
import functools

import jax
import jax.numpy as jnp
from jax import lax
from jax.experimental import pallas as pl
from jax.experimental.pallas import tpu as pltpu
from jax.experimental.pallas import tpu_sc as plsc

NUM_CORES = 2
NUM_SUBCORES = 16
NUM_WORKERS = NUM_CORES * NUM_SUBCORES

V = 1000000
D = 64
VMAIN = 999936
NCOLS = 7812
COLS_PER_TILE = 245

CHUNK = 128
RING = 4
LAG = 2


def _iota16():
    return lax.iota(jnp.int32, 16)


def _diag_vectors():
    lanes = _iota16()
    return [(lanes + dd) & 15 for dd in range(16)], lanes


def _transpose_block(src_ref, dst_ref, n_s, n_d, rots, lanes):
    def srow(si, carry):
        svec = si * 16 + lanes
        for di in range(n_d // 16):
            xs = []
            for dd in range(16):
                xs.append(plsc.load_gather(src_ref, [di * 16 + rots[dd], svec]))
            for dd in range(16):
                plsc.store_scatter(dst_ref, [svec, di * 16 + rots[dd]], xs[dd])
        return carry

    lax.fori_loop(0, n_s // 16, srow, 0)


@jax.jit
def _sc_transpose(wt, tail):
    mesh = plsc.VectorSubcoreMesh(core_axis_name="c", subcore_axis_name="s")

    @functools.partial(
        pl.kernel,
        mesh=mesh,
        out_type=jax.ShapeDtypeStruct((V, 128), jnp.float32),
        scratch_types=[
            pltpu.VMEM((2, D, 128), jnp.float32),
            pltpu.VMEM((2, 128, 128), jnp.float32),
            pltpu.VMEM((D, D), jnp.float32),
            pltpu.SemaphoreType.DMA((2,)),
            pltpu.SemaphoreType.DMA((2,)),
        ],
        compiler_params=pltpu.CompilerParams(use_tc_tiling_on_sc=True, needs_layout_passes=False),
    )
    def k(wt_hbm, tail_hbm, out_hbm, blk_v, obuf_v, tail_v, isem, osem):
        wid = lax.axis_index("s") * NUM_CORES + lax.axis_index("c")
        rots, lanes0 = _diag_vectors()

        def col_of(kk):
            return jnp.minimum(wid + 32 * kk, NCOLS - 1)

        def fire_in(c, p):
            for r in range(8):
                pltpu.async_copy(
                    wt_hbm.at[pl.ds(r * 8, 8), pl.ds(c * 128, 128)],
                    blk_v.at[p, pl.ds(r * 8, 8)],
                    isem.at[p],
                )

        def wait_in(c, p):
            for r in range(8):
                pltpu.make_async_copy(
                    wt_hbm.at[pl.ds(r * 8, 8), pl.ds(c * 128, 128)],
                    blk_v.at[p, pl.ds(r * 8, 8)],
                    isem.at[p],
                ).wait()

        def fire_out(c, p):
            pltpu.async_copy(
                obuf_v.at[p], out_hbm.at[pl.ds(c * 128, 128)], osem.at[p]
            )

        def wait_out(c, p):
            pltpu.make_async_copy(
                obuf_v.at[p], out_hbm.at[pl.ds(c * 128, 128)], osem.at[p]
            ).wait()

        fire_in(col_of(0), 0)
        fire_in(col_of(1), 1)

        def body(kk, carry):
            p = kk & 1
            c = col_of(kk)
            wait_in(c, p)
            pl.when(kk >= 2)(lambda: wait_out(col_of(kk - 2), p))
            _transpose_block(blk_v.at[p], obuf_v.at[p], 128, D, rots, lanes0)
            fire_out(c, p)
            pl.when(kk <= 242)(lambda: fire_in(col_of(kk + 2), p))
            return carry

        lax.fori_loop(0, 245, body, 0)
        wait_out(col_of(243), 1)
        wait_out(col_of(244), 0)

        pltpu.sync_copy(tail_hbm, tail_v)

        def tail_sub(t, carry):
            svec = (t // 4) * 16 + lanes0
            for dd in range(16):
                dvec = (t % 4) * 16 + rots[dd]
                x = plsc.load_gather(tail_v, [dvec, svec])
                plsc.store_scatter(obuf_v.at[0], [svec, dvec], x)
            return carry

        lax.fori_loop(0, 16, tail_sub, 0)
        pltpu.sync_copy(
            obuf_v.at[0, pl.ds(0, D)], out_hbm.at[pl.ds(VMAIN, D)]
        )

    return k(wt, tail)


@functools.partial(jax.jit, static_argnums=(2, 3))
def _sc_gather(table, flat_idx, b_per_w, n_chunks):
    total = flat_idx.shape[0]
    mesh = plsc.VectorSubcoreMesh(core_axis_name="c", subcore_axis_name="s")

    @functools.partial(
        pl.kernel,
        mesh=mesh,
        out_type=jax.ShapeDtypeStruct((total, 128), jnp.float32),
        scratch_types=[
            pltpu.VMEM((b_per_w,), jnp.int32),
            pltpu.VMEM((RING, CHUNK, 128), jnp.float32),
            pltpu.SemaphoreType.DMA((RING,)),
            pltpu.SemaphoreType.DMA((RING,)),
        ],
        compiler_params=pltpu.CompilerParams(use_tc_tiling_on_sc=True, needs_layout_passes=False),
    )
    def k(table_hbm, idx_hbm, out_hbm, idx_v, rows_v, gsem, wsem):
        wid = lax.axis_index("s") * NUM_CORES + lax.axis_index("c")
        base = wid * b_per_w
        pltpu.sync_copy(idx_hbm.at[pl.ds(base, b_per_w)], idx_v)

        def start_gather(c, s):
            pltpu.async_copy(
                table_hbm.at[idx_v.at[pl.ds(c * CHUNK, CHUNK)]],
                rows_v.at[s],
                gsem.at[s],
            )

        def wait_gather(c, s):
            pltpu.make_async_copy(
                table_hbm.at[idx_v.at[pl.ds(c * CHUNK, CHUNK)]],
                rows_v.at[s],
                gsem.at[s],
            ).wait()

        def start_write(c, s):
            pltpu.async_copy(
                rows_v.at[s],
                out_hbm.at[pl.ds(base + c * CHUNK, CHUNK)],
                wsem.at[s],
            )

        def wait_write(c, s):
            pltpu.make_async_copy(
                rows_v.at[s],
                out_hbm.at[pl.ds(base + c * CHUNK, CHUNK)],
                wsem.at[s],
            ).wait()

        for i in range(RING):
            start_gather(i, i)
            if i >= LAG:
                j = i - LAG
                wait_gather(j, j)
                start_write(j, j)

        def body(kk, carry):
            i0 = kk * RING
            for b in range(RING):
                sw = (b - LAG) % RING
                wait_write(i0 + b - RING, b)
                start_gather(i0 + b, b)
                wait_gather(i0 + b - LAG, sw)
                start_write(i0 + b - LAG, sw)
            return carry

        lax.fori_loop(1, n_chunks // RING, body, 0)

        for b in range(LAG):
            c = n_chunks - LAG + b
            s = c % RING
            wait_gather(c, s)
            start_write(c, s)
        for b in range(RING):
            wait_write(n_chunks - RING + b, (n_chunks - RING + b) % RING)

    return k(table, flat_idx)


def kernel(token_ids, weight):
    b = token_ids.size
    flat = token_ids.reshape((b,)).astype(jnp.int32)
    wt = weight.T
    tail = weight[VMAIN:].T
    table = _sc_transpose(wt, tail)
    b_per_w = b // NUM_WORKERS
    n_chunks = b_per_w // CHUNK
    y = _sc_gather(table, flat, b_per_w, n_chunks)
    return y[:, :D].reshape(token_ids.shape + (D,))


# speedup vs baseline: 1.9920x; 1.1590x over previous
import functools

import jax
import jax.numpy as jnp
from jax import lax
from jax.experimental import pallas as pl
from jax.experimental.pallas import tpu as pltpu
from jax.experimental.pallas import tpu_sc as plsc

NUM_CORES = 2
NUM_SUBCORES = 16
NUM_WORKERS = NUM_CORES * NUM_SUBCORES

V = 1000000
D = 64
VMAIN = 999936
NCOLS = 7812
COLS_PER_TILE = 245

CHUNK = 128
RING = 4
LAG = 2


def _iota16():
    return lax.iota(jnp.int32, 16)


def _diag_vectors():
    lanes = _iota16()
    return [(lanes + dd) & 15 for dd in range(16)], lanes


def _transpose_block(src_ref, dst_ref, n_s, n_d, rots, lanes):
    def srow(si, carry):
        svec = si * 16 + lanes
        for di in range(n_d // 16):
            xs = []
            for dd in range(16):
                xs.append(plsc.load_gather(src_ref, [di * 16 + rots[dd], svec]))
            for dd in range(16):
                plsc.store_scatter(dst_ref, [svec, di * 16 + rots[dd]], xs[dd])
        return carry

    lax.fori_loop(0, n_s // 16, srow, 0)


@jax.jit
def _sc_transpose(wt, tail):
    mesh = plsc.VectorSubcoreMesh(core_axis_name="c", subcore_axis_name="s")

    @functools.partial(
        pl.kernel,
        mesh=mesh,
        out_type=jax.ShapeDtypeStruct((V, 128), jnp.float32),
        scratch_types=[
            pltpu.VMEM((2, D, 128), jnp.float32),
            pltpu.VMEM((2, 128, 128), jnp.float32),
            pltpu.VMEM((D, D), jnp.float32),
            pltpu.SemaphoreType.DMA((2,)),
            pltpu.SemaphoreType.DMA((2,)),
        ],
        compiler_params=pltpu.CompilerParams(use_tc_tiling_on_sc=True, needs_layout_passes=False),
    )
    def k(wt_hbm, tail_hbm, out_hbm, blk_v, obuf_v, tail_v, isem, osem):
        wid = lax.axis_index("s") * NUM_CORES + lax.axis_index("c")
        rots, lanes0 = _diag_vectors()

        def col_of(kk):
            return jnp.minimum(wid + 32 * kk, NCOLS - 1)

        def fire_in(c, p):
            for r in range(8):
                pltpu.async_copy(
                    wt_hbm.at[pl.ds(r * 8, 8), pl.ds(c * 128, 128)],
                    blk_v.at[p, pl.ds(r * 8, 8)],
                    isem.at[p],
                )

        def wait_in(c, p):
            for r in range(8):
                pltpu.make_async_copy(
                    wt_hbm.at[pl.ds(r * 8, 8), pl.ds(c * 128, 128)],
                    blk_v.at[p, pl.ds(r * 8, 8)],
                    isem.at[p],
                ).wait()

        def fire_out(c, p):
            pltpu.async_copy(
                obuf_v.at[p], out_hbm.at[pl.ds(c * 128, 128)], osem.at[p]
            )

        def wait_out(c, p):
            pltpu.make_async_copy(
                obuf_v.at[p], out_hbm.at[pl.ds(c * 128, 128)], osem.at[p]
            ).wait()

        fire_in(col_of(0), 0)
        fire_in(col_of(1), 1)

        def body(kk, carry):
            p = kk & 1
            c = col_of(kk)
            wait_in(c, p)
            pl.when(kk >= 2)(lambda: wait_out(col_of(kk - 2), p))
            _transpose_block(blk_v.at[p], obuf_v.at[p], 128, D, rots, lanes0)
            fire_out(c, p)
            pl.when(kk <= 242)(lambda: fire_in(col_of(kk + 2), p))
            return carry

        lax.fori_loop(0, 245, body, 0)
        wait_out(col_of(243), 1)
        wait_out(col_of(244), 0)

        pltpu.sync_copy(tail_hbm, tail_v)

        def tail_sub(t, carry):
            svec = (t // 4) * 16 + lanes0
            for dd in range(16):
                dvec = (t % 4) * 16 + rots[dd]
                x = plsc.load_gather(tail_v, [dvec, svec])
                plsc.store_scatter(obuf_v.at[0], [svec, dvec], x)
            return carry

        lax.fori_loop(0, 16, tail_sub, 0)
        pltpu.sync_copy(
            obuf_v.at[0, pl.ds(0, D)], out_hbm.at[pl.ds(VMAIN, D)]
        )

    return k(wt, tail)


BATCH = 4096
SEQ = 200
B_PER_TILE = BATCH // NUM_WORKERS


@jax.jit
def _sc_gather_t(table, flat_idx):
    mesh = plsc.VectorSubcoreMesh(core_axis_name="c", subcore_axis_name="s")

    @functools.partial(
        pl.kernel,
        mesh=mesh,
        out_type=jax.ShapeDtypeStruct((SEQ, 8, 32, 8, 128), jnp.float32),
        scratch_types=[
            pltpu.VMEM((B_PER_TILE * SEQ,), jnp.int32),
            pltpu.VMEM((B_PER_TILE * SEQ,), jnp.int32),
            pltpu.VMEM((2, 128, 128), jnp.float32),
            pltpu.VMEM((2, D, 128), jnp.float32),
            pltpu.SemaphoreType.DMA((2,)),
            pltpu.SemaphoreType.DMA((2,)),
        ],
        compiler_params=pltpu.CompilerParams(use_tc_tiling_on_sc=True, needs_layout_passes=False),
    )
    def k(table_hbm, idx_hbm, out_hbm, idx_v, idxt_v, rows_v, img_v,
          gsem, wsem):
        wid = lax.axis_index("s") * NUM_CORES + lax.axis_index("c")
        base = wid * (B_PER_TILE * SEQ)
        pltpu.sync_copy(idx_hbm.at[pl.ds(base, B_PER_TILE * SEQ)], idx_v)
        lanes = _iota16()
        rots = [(lanes + dd) & 15 for dd in range(16)]
        l200 = lanes * SEQ

        def idx_t(s, carry):
            for bl0 in range(0, 128, 16):
                x = plsc.load_gather(idx_v, [s + bl0 * SEQ + l200])
                plsc.store_scatter(idxt_v, [s * 128 + bl0 + lanes], x)
            return carry

        lax.fori_loop(0, SEQ, idx_t, 0)

        def fire_gather(s, p):
            pltpu.async_copy(
                table_hbm.at[idxt_v.at[pl.ds(s * 128, 128)]],
                rows_v.at[p],
                gsem.at[p],
            )

        def wait_gather(s, p):
            pltpu.make_async_copy(
                table_hbm.at[idxt_v.at[pl.ds(s * 128, 128)]],
                rows_v.at[p],
                gsem.at[p],
            ).wait()

        def fire_img(s, p):
            for r in range(8):
                pltpu.async_copy(
                    img_v.at[p, pl.ds(r * 8, 8)],
                    out_hbm.at[s, r, wid],
                    wsem.at[p],
                )

        def wait_img(s, p):
            for r in range(8):
                pltpu.make_async_copy(
                    img_v.at[p, pl.ds(r * 8, 8)],
                    out_hbm.at[s, r, wid],
                    wsem.at[p],
                ).wait()

        def transpose_slab(p):
            src = rows_v.at[p]
            dst = img_v.at[p]

            def blblock(bi, carry):
                blvec = bi * 16 + lanes
                for di in range(4):
                    xs = []
                    for dd in range(16):
                        xs.append(
                            plsc.load_gather(src, [blvec, di * 16 + rots[dd]])
                        )
                    for dd in range(16):
                        plsc.store_scatter(
                            dst, [di * 16 + rots[dd], blvec], xs[dd]
                        )
                return carry

            lax.fori_loop(0, 8, blblock, 0)

        fire_gather(0, 0)
        fire_gather(1, 1)

        def body(s, carry):
            p = s & 1
            wait_gather(s, p)
            pl.when(s >= 2)(lambda: wait_img(s - 2, p))
            transpose_slab(p)
            fire_img(s, p)
            pl.when(s <= SEQ - 3)(lambda: fire_gather(s + 2, p))
            return carry

        lax.fori_loop(0, SEQ, body, 0)
        wait_img(SEQ - 2, 0)
        wait_img(SEQ - 1, 1)

    return k(table, flat_idx)


def kernel(token_ids, weight):
    b = token_ids.size
    flat = token_ids.reshape((b,)).astype(jnp.int32)
    wt = weight.T
    tail = weight[VMAIN:].T
    table = _sc_transpose(wt, tail)
    y = _sc_gather_t(table, flat)
    out = y.transpose(2, 4, 0, 1, 3).reshape(BATCH, SEQ, D)
    return out.reshape(token_ids.shape + (D,))


# speedup vs baseline: 2.1885x; 1.0986x over previous
import functools

import jax
import jax.numpy as jnp
from jax import lax
from jax.experimental import pallas as pl
from jax.experimental.pallas import tpu as pltpu
from jax.experimental.pallas import tpu_sc as plsc

NUM_CORES = 2
NUM_SUBCORES = 16
NUM_WORKERS = NUM_CORES * NUM_SUBCORES

V = 1000000
D = 64
VMAIN = 999936
NCOLS = 7812
COLS_PER_TILE = 245

CHUNK = 128
RING = 4
LAG = 2


def _iota16():
    return lax.iota(jnp.int32, 16)


def _diag_vectors():
    lanes = _iota16()
    return [(lanes + dd) & 15 for dd in range(16)], lanes


def _transpose_block_flat(src_ref, dst_flat, n_s, n_d, rots, lanes, wvs):
    def srow(si, carry):
        svec = si * 16 + lanes
        for di in range(n_d // 16):
            base_w = si * 16 * n_d + di * 16
            xs = []
            for dd in range(16):
                xs.append(plsc.load_gather(src_ref, [di * 16 + rots[dd], svec]))
            for dd in range(16):
                plsc.store_scatter(dst_flat, [base_w + wvs[dd]], xs[dd])
        return carry

    lax.fori_loop(0, n_s // 16, srow, 0)


@jax.jit
def _sc_transpose(wt, tail):
    mesh = plsc.VectorSubcoreMesh(core_axis_name="c", subcore_axis_name="s")

    @functools.partial(
        pl.kernel,
        mesh=mesh,
        out_type=jax.ShapeDtypeStruct((V * D,), jnp.float32),
        scratch_types=[
            pltpu.VMEM((2, D, 128), jnp.float32),
            pltpu.VMEM((2 * 128 * D,), jnp.float32),
            pltpu.VMEM((D, D), jnp.float32),
            pltpu.SemaphoreType.DMA((2,)),
            pltpu.SemaphoreType.DMA((2,)),
        ],
        compiler_params=pltpu.CompilerParams(use_tc_tiling_on_sc=True, needs_layout_passes=False),
    )
    def k(wt_hbm, tail_hbm, out_hbm, blk_v, obuf_v, tail_v, isem, osem):
        wid = lax.axis_index("s") * NUM_CORES + lax.axis_index("c")
        rots, lanes0 = _diag_vectors()
        wvs = [lanes0 * D + rots[dd] for dd in range(16)]

        def col_of(kk):
            return jnp.minimum(wid + 32 * kk, NCOLS - 1)

        def fire_in(c, p):
            for r in range(8):
                pltpu.async_copy(
                    wt_hbm.at[pl.ds(r * 8, 8), pl.ds(c * 128, 128)],
                    blk_v.at[p, pl.ds(r * 8, 8)],
                    isem.at[p],
                )

        def wait_in(c, p):
            for r in range(8):
                pltpu.make_async_copy(
                    wt_hbm.at[pl.ds(r * 8, 8), pl.ds(c * 128, 128)],
                    blk_v.at[p, pl.ds(r * 8, 8)],
                    isem.at[p],
                ).wait()

        def fire_out(c, p):
            pltpu.async_copy(
                obuf_v.at[pl.ds(p * 8192, 8192)],
                out_hbm.at[pl.ds(c * 8192, 8192)], osem.at[p]
            )

        def wait_out(c, p):
            pltpu.make_async_copy(
                obuf_v.at[pl.ds(p * 8192, 8192)],
                out_hbm.at[pl.ds(c * 8192, 8192)], osem.at[p]
            ).wait()

        fire_in(col_of(0), 0)
        fire_in(col_of(1), 1)

        def body(kk, carry):
            p = kk & 1
            c = col_of(kk)
            wait_in(c, p)
            pl.when(kk >= 2)(lambda: wait_out(col_of(kk - 2), p))
            _transpose_block_flat(blk_v.at[p], obuf_v.at[pl.ds(p * 8192, 8192)], 128, D, rots, lanes0, wvs)
            fire_out(c, p)
            pl.when(kk <= 242)(lambda: fire_in(col_of(kk + 2), p))
            return carry

        lax.fori_loop(0, 245, body, 0)
        wait_out(col_of(243), 1)
        wait_out(col_of(244), 0)

        pltpu.sync_copy(tail_hbm, tail_v)

        def tail_sub(t, carry):
            si = t // 4
            svec = si * 16 + lanes0
            for dd in range(16):
                dvec = (t % 4) * 16 + rots[dd]
                x = plsc.load_gather(tail_v, [dvec, svec])
                plsc.store_scatter(
                    obuf_v.at[pl.ds(0, 8192)],
                    [si * 16 * D + (t % 4) * 16 + wvs[dd]],
                    x,
                )
            return carry

        lax.fori_loop(0, 16, tail_sub, 0)
        pltpu.sync_copy(
            obuf_v.at[pl.ds(0, D * D)],
            out_hbm.at[pl.ds(VMAIN * D, D * D)],
        )

    return k(wt, tail)


BATCH = 4096
SEQ = 200
B_PER_TILE = BATCH // NUM_WORKERS


@jax.jit
def _sc_gather_t(table, flat_idx):
    mesh = plsc.VectorSubcoreMesh(core_axis_name="c", subcore_axis_name="s")

    @functools.partial(
        pl.kernel,
        mesh=mesh,
        out_type=jax.ShapeDtypeStruct((SEQ * 8 * 32 * 8, 128), jnp.float32),
        scratch_types=[
            pltpu.VMEM((B_PER_TILE * SEQ,), jnp.int32),
            pltpu.VMEM((B_PER_TILE * SEQ,), jnp.int32),
            pltpu.VMEM((2 * 128, D), jnp.float32),
            pltpu.VMEM((2 * D, 128), jnp.float32),
            pltpu.SemaphoreType.DMA((2,)),
            pltpu.SemaphoreType.DMA((2,)),
        ],
        compiler_params=pltpu.CompilerParams(use_tc_tiling_on_sc=False, needs_layout_passes=False),
    )
    def k(table_hbm, idx_hbm, out_hbm, idx_v, idxt_v, rows_v, img_v,
          gsem, wsem):
        wid = lax.axis_index("s") * NUM_CORES + lax.axis_index("c")
        base = wid * (B_PER_TILE * SEQ)
        pltpu.sync_copy(idx_hbm.at[pl.ds(base, B_PER_TILE * SEQ)], idx_v)
        lanes = _iota16()
        rots = [(lanes + dd) & 15 for dd in range(16)]
        l200 = lanes * SEQ

        def idx_t(s, carry):
            for bl0 in range(0, 128, 16):
                x = plsc.load_gather(idx_v, [s + bl0 * SEQ + l200])
                plsc.store_scatter(idxt_v, [s * 128 + bl0 + lanes], x)
            return carry

        lax.fori_loop(0, SEQ, idx_t, 0)

        def fire_gather(s, p):
            pltpu.async_copy(
                table_hbm.at[idxt_v.at[pl.ds(s * 128, 128)]],
                rows_v.at[pl.ds(p * 128, 128)],
                gsem.at[p],
            )

        def wait_gather(s, p):
            pltpu.make_async_copy(
                table_hbm.at[idxt_v.at[pl.ds(s * 128, 128)]],
                rows_v.at[pl.ds(p * 128, 128)],
                gsem.at[p],
            ).wait()

        def fire_img(s, p):
            for r in range(8):
                row0 = ((s * 8 + r) * 32 + wid) * 8
                pltpu.async_copy(
                    img_v.at[pl.ds(p * D + r * 8, 8)],
                    out_hbm.at[pl.ds(row0, 8)],
                    wsem.at[p],
                )

        def wait_img(s, p):
            for r in range(8):
                row0 = ((s * 8 + r) * 32 + wid) * 8
                pltpu.make_async_copy(
                    img_v.at[pl.ds(p * D + r * 8, 8)],
                    out_hbm.at[pl.ds(row0, 8)],
                    wsem.at[p],
                ).wait()

        def transpose_slab(p):
            src = rows_v.at[pl.ds(p * 128, 128)]
            dst = img_v.at[pl.ds(p * D, D)]

            def blblock(bi, carry):
                blvec = bi * 16 + lanes
                for di in range(4):
                    xs = []
                    for dd in range(16):
                        xs.append(
                            plsc.load_gather(src, [blvec, di * 16 + rots[dd]])
                        )
                    for dd in range(16):
                        plsc.store_scatter(
                            dst, [di * 16 + rots[dd], blvec], xs[dd]
                        )
                return carry

            lax.fori_loop(0, 8, blblock, 0)

        fire_gather(0, 0)
        fire_gather(1, 1)

        def body(s, carry):
            p = s & 1
            wait_gather(s, p)
            pl.when(s >= 2)(lambda: wait_img(s - 2, p))
            transpose_slab(p)
            fire_img(s, p)
            pl.when(s <= SEQ - 3)(lambda: fire_gather(s + 2, p))
            return carry

        lax.fori_loop(0, SEQ, body, 0)
        wait_img(SEQ - 2, 0)
        wait_img(SEQ - 1, 1)

    return k(table, flat_idx)


def kernel(token_ids, weight):
    b = token_ids.size
    flat = token_ids.reshape((b,)).astype(jnp.int32)
    wt = weight.T
    tail = weight[VMAIN:].T
    table = _sc_transpose(wt, tail).reshape(V, D)
    y = _sc_gather_t(table, flat).reshape(SEQ, 8, 32, 8, 128)
    out = y.transpose(2, 4, 0, 1, 3).reshape(BATCH, SEQ, D)
    return out.reshape(token_ids.shape + (D,))
